# Initial kernel scaffold; baseline (speedup 1.0000x reference)
import jax
import jax.numpy as jnp
from jax.experimental import pallas as pl


def kernel(x, edge_index, batch, W1, b1, W2, b2, Wp, bp):
    raise NotImplementedError("write your pallas kernel here")


# speedup vs baseline: 8.4027x; 8.4027x over previous
import functools

import jax
import jax.numpy as jnp
from jax import lax
from jax.experimental import pallas as pl
from jax.experimental.pallas import tpu as pltpu
from jax.experimental.pallas import tpu_sc as plsc

N_NODES = 10000
NPAD = 10240
N_EDGES = 320000
N_GRAPHS = 64
IN_DIM = 128
HID_DIM = 256
OUT_DIM = 128
HALF = HID_DIM // 2

NB = 10
BLK = NPAD // NB

NSUB = 16
NCORE = 2
CHUNK = 40
EPS = N_EDGES // NSUB
NWIN = EPS // CHUNK
ROWS_PER_SUB = NPAD // NSUB
WCH = 32
NWCH = ROWS_PER_SUB // WCH

_mesh = plsc.VectorSubcoreMesh(core_axis_name="c", subcore_axis_name="s")


@jax.jit
def _segsum(y0, y1, srcf, dstf, rif, zr):
    @functools.partial(
        pl.kernel,
        mesh=_mesh,
        out_type=(jax.ShapeDtypeStruct((NPAD, HALF), jnp.float32),
                  jax.ShapeDtypeStruct((NPAD, HALF), jnp.float32)),
        scratch_types=[
            pltpu.VMEM((CHUNK,), jnp.int32),
            pltpu.VMEM((CHUNK,), jnp.int32),
            pltpu.VMEM((CHUNK,), jnp.int32),
            pltpu.VMEM((CHUNK,), jnp.int32),
            pltpu.VMEM((CHUNK, HALF), jnp.float32),
            pltpu.VMEM((CHUNK, HALF), jnp.float32),
            pltpu.VMEM((WCH,), jnp.int32),
            pltpu.VMEM((WCH, HALF), jnp.float32),
            pltpu.VMEM_SHARED((NPAD, HALF), jnp.float32),
            pltpu.SemaphoreType.DMA,
            pltpu.SemaphoreType.DMA,
            pltpu.SemaphoreType.DMA,
            pltpu.SemaphoreType.DMA,
            pltpu.SemaphoreType.DMA,
            pltpu.SemaphoreType.DMA,
        ],
    )
    def k(y0_h, y1_h, src_h, dst_h, ri_h, zr_h, out0_h, out1_h,
          isa, isb, ida, idb, rows_a, rows_b, ribuf, zob, acc_sh,
          s1, s2, s3, s4, sa, sb):
        c = lax.axis_index("c")
        s = lax.axis_index("s")
        ebase = s * EPS

        pltpu.sync_copy(zr_h, zob)
        for t in range(NWCH):
            pltpu.sync_copy(ri_h.at[pl.ds(s * ROWS_PER_SUB + t * WCH, WCH)],
                            ribuf)
            pltpu.sync_copy(zob, acc_sh.at[ribuf])
        plsc.subcore_barrier()

        def icp(buf, ref1d, j, sem):
            return pltpu.make_async_copy(
                ref1d.at[pl.ds(ebase + j * CHUNK, CHUNK)], buf, sem)

        def edge_loop(y_h):
            icp(isa, src_h, 0, s1).start()
            icp(ida, dst_h, 0, s2).start()

            @pl.loop(0, NWIN, step=2)
            def _(j):
                icp(isa, src_h, j, s1).wait()
                icp(ida, dst_h, j, s2).wait()
                ga = pltpu.async_copy(y_h.at[isa], rows_a, sa)
                icp(isb, src_h, j + 1, s3).start()
                icp(idb, dst_h, j + 1, s4).start()
                icp(isb, src_h, j + 1, s3).wait()
                icp(idb, dst_h, j + 1, s4).wait()
                gb = pltpu.async_copy(y_h.at[isb], rows_b, sb)
                ga.wait()
                pltpu.sync_copy(rows_a, acc_sh.at[ida], add=True)

                @pl.when(j + 2 < NWIN)
                def _():
                    icp(isa, src_h, j + 2, s1).start()
                    icp(ida, dst_h, j + 2, s2).start()

                gb.wait()
                pltpu.sync_copy(rows_b, acc_sh.at[idb], add=True)

        @pl.when(c == 0)
        def _():
            edge_loop(y0_h)

        @pl.when(c == 1)
        def _():
            edge_loop(y1_h)

        plsc.subcore_barrier()

        def readout(out_h):
            for t in range(NWCH):
                base = s * ROWS_PER_SUB + t * WCH
                pltpu.sync_copy(ri_h.at[pl.ds(base, WCH)], ribuf)
                pltpu.sync_copy(acc_sh.at[ribuf], zob)
                pltpu.sync_copy(zob, out_h.at[pl.ds(base, WCH)])

        @pl.when(c == 0)
        def _():
            readout(out0_h)

        @pl.when(c == 1)
        def _():
            readout(out1_h)

    return k(y0, y1, srcf, dstf, rif, zr)


def _stage_a_body(d0_ref, d1_ref, x_ref, w_ref, y0_ref, y1_ref, dinv_ref):
    deg = d0_ref[:, 0:1] + 0.0 * d1_ref[:, 0:1] + 1.0
    dinv = lax.rsqrt(deg)
    xw = jnp.dot(x_ref[...], w_ref[...], preferred_element_type=jnp.float32)
    yv = xw * dinv
    y0_ref[...] = yv[:, :HALF]
    y1_ref[...] = yv[:, HALF:]
    dinv_ref[...] = jnp.broadcast_to(dinv, (BLK, 16))


@jax.jit
def _stage_a(deg0, deg1, x, W1):
    return pl.pallas_call(
        _stage_a_body,
        grid=(NB,),
        in_specs=[
            pl.BlockSpec((BLK, HALF), lambda i: (i, 0)),
            pl.BlockSpec((BLK, HALF), lambda i: (i, 0)),
            pl.BlockSpec((BLK, IN_DIM), lambda i: (i, 0)),
            pl.BlockSpec((IN_DIM, HID_DIM), lambda i: (0, 0)),
        ],
        out_specs=[
            pl.BlockSpec((BLK, HALF), lambda i: (i, 0)),
            pl.BlockSpec((BLK, HALF), lambda i: (i, 0)),
            pl.BlockSpec((BLK, 16), lambda i: (i, 0)),
        ],
        out_shape=[
            jax.ShapeDtypeStruct((NPAD, HALF), jnp.float32),
            jax.ShapeDtypeStruct((NPAD, HALF), jnp.float32),
            jax.ShapeDtypeStruct((NPAD, 16), jnp.float32),
        ],
    )(deg0, deg1, x, W1)


def _stage_b_body(s0_ref, s1_ref, y0_ref, y1_ref, dinv_ref, b_ref, w_ref,
                  o0_ref, o1_ref):
    agg = jnp.concatenate([s0_ref[...], s1_ref[...]], axis=1)
    y = jnp.concatenate([y0_ref[...], y1_ref[...]], axis=1)
    dinv = dinv_ref[:, 0:1]
    h = jnp.maximum(dinv * (agg + y) + b_ref[...], 0.0)
    z = jnp.dot(h, w_ref[...], preferred_element_type=jnp.float32)
    y2 = z * dinv
    o0_ref[...] = y2[:, :HALF]
    o1_ref[...] = y2[:, HALF:]


@jax.jit
def _stage_b(S0, S1, y0, y1, dinv16, b1, W2):
    return pl.pallas_call(
        _stage_b_body,
        grid=(NB,),
        in_specs=[
            pl.BlockSpec((BLK, HALF), lambda i: (i, 0)),
            pl.BlockSpec((BLK, HALF), lambda i: (i, 0)),
            pl.BlockSpec((BLK, HALF), lambda i: (i, 0)),
            pl.BlockSpec((BLK, HALF), lambda i: (i, 0)),
            pl.BlockSpec((BLK, 16), lambda i: (i, 0)),
            pl.BlockSpec((1, HID_DIM), lambda i: (0, 0)),
            pl.BlockSpec((HID_DIM, HID_DIM), lambda i: (0, 0)),
        ],
        out_specs=[
            pl.BlockSpec((BLK, HALF), lambda i: (i, 0)),
            pl.BlockSpec((BLK, HALF), lambda i: (i, 0)),
        ],
        out_shape=[
            jax.ShapeDtypeStruct((NPAD, HALF), jnp.float32),
            jax.ShapeDtypeStruct((NPAD, HALF), jnp.float32),
        ],
    )(S0, S1, y0, y1, dinv16, b1, W2)


def _stage_c_body(s0_ref, s1_ref, y0_ref, y1_ref, dinv_ref, b_ref, batch_ref,
                  wp_ref, bp_ref, o_ref, sums, cnt):
    i = pl.program_id(0)

    @pl.when(i == 0)
    def _():
        sums[...] = jnp.zeros((N_GRAPHS, HID_DIM), jnp.float32)
        cnt[...] = jnp.zeros((N_GRAPHS, 128), jnp.float32)

    agg = jnp.concatenate([s0_ref[...], s1_ref[...]], axis=1)
    y = jnp.concatenate([y0_ref[...], y1_ref[...]], axis=1)
    dinv = dinv_ref[:, 0:1]
    h = jnp.maximum(dinv * (agg + y) + b_ref[...], 0.0)
    gids = lax.broadcasted_iota(jnp.int32, (N_GRAPHS, BLK), 0)
    mask = (batch_ref[0] == gids).astype(jnp.float32)
    sums[...] += jnp.dot(mask, h, preferred_element_type=jnp.float32)
    cnt[...] += jnp.broadcast_to(
        jnp.sum(mask, axis=1, keepdims=True), (N_GRAPHS, 128))

    @pl.when(i == NB - 1)
    def _():
        hg = sums[...] / jnp.maximum(cnt[:, 0:1], 1.0)
        o_ref[...] = (
            jnp.dot(hg, wp_ref[...], preferred_element_type=jnp.float32)
            + bp_ref[...])


@jax.jit
def _stage_c(S0, S1, y0, y1, dinv16, b2, batch3, Wp, bp):
    return pl.pallas_call(
        _stage_c_body,
        grid=(NB,),
        in_specs=[
            pl.BlockSpec((BLK, HALF), lambda i: (i, 0)),
            pl.BlockSpec((BLK, HALF), lambda i: (i, 0)),
            pl.BlockSpec((BLK, HALF), lambda i: (i, 0)),
            pl.BlockSpec((BLK, HALF), lambda i: (i, 0)),
            pl.BlockSpec((BLK, 16), lambda i: (i, 0)),
            pl.BlockSpec((1, HID_DIM), lambda i: (0, 0)),
            pl.BlockSpec((1, 1, BLK), lambda i: (i, 0, 0)),
            pl.BlockSpec((HID_DIM, OUT_DIM), lambda i: (0, 0)),
            pl.BlockSpec((1, OUT_DIM), lambda i: (0, 0)),
        ],
        out_specs=pl.BlockSpec((N_GRAPHS, OUT_DIM), lambda i: (0, 0)),
        out_shape=jax.ShapeDtypeStruct((N_GRAPHS, OUT_DIM), jnp.float32),
        scratch_shapes=[
            pltpu.VMEM((N_GRAPHS, HID_DIM), jnp.float32),
            pltpu.VMEM((N_GRAPHS, 128), jnp.float32),
        ],
    )(S0, S1, y0, y1, dinv16, b2, batch3, Wp, bp)


def kernel(x, edge_index, batch, W1, b1, W2, b2, Wp, bp):
    src = edge_index[0].astype(jnp.int32)
    dst = edge_index[1].astype(jnp.int32)
    xp = jnp.pad(x, ((0, NPAD - N_NODES), (0, 0)))
    batch3 = jnp.pad(batch.astype(jnp.int32), (0, NPAD - N_NODES),
                     constant_values=N_GRAPHS).reshape(NB, 1, BLK)

    rif = jnp.arange(NPAD, dtype=jnp.int32)
    zr = jnp.zeros((WCH, HALF), jnp.float32)
    ones_y = jnp.ones((NPAD, HALF), jnp.float32)

    deg0, deg1 = _segsum(ones_y, ones_y, dst, dst, rif, zr)
    y0, y1, dinv16 = _stage_a(deg0, deg1, xp, W1)
    S0, S1 = _segsum(y0, y1, src, dst, rif, zr)
    z0, z1 = _stage_b(S0, S1, y0, y1, dinv16, b1.reshape(1, HID_DIM), W2)
    T0, T1 = _segsum(z0, z1, src, dst, rif, zr)
    return _stage_c(T0, T1, z0, z1, dinv16, b2.reshape(1, HID_DIM), batch3,
                    Wp, bp.reshape(1, OUT_DIM))


# speedup vs baseline: 13.4123x; 1.5962x over previous
import functools

import jax
import jax.numpy as jnp
from jax import lax
from jax.experimental import pallas as pl
from jax.experimental.pallas import tpu as pltpu
from jax.experimental.pallas import tpu_sc as plsc

N_NODES = 10000
NPAD = 10240
N_EDGES = 320000
N_GRAPHS = 64
IN_DIM = 128
HID_DIM = 256
OUT_DIM = 128
HALF = HID_DIM // 2

NB = 10
BLK = NPAD // NB

NSUB = 16
NCORE = 2
CHUNK = 80
EPS = N_EDGES // NSUB
NWIN = EPS // CHUNK
ROWS_PER_SUB = NPAD // NSUB
WCH = 16
NWCH = ROWS_PER_SUB // WCH
DEPS = N_EDGES // (NCORE * NSUB)
DCH = 40
DNW = DEPS // DCH

_mesh = plsc.VectorSubcoreMesh(core_axis_name="c", subcore_axis_name="s")


@jax.jit
def _segsum(y0, y1, srcf, dstf, rif, zr):
    @functools.partial(
        pl.kernel,
        mesh=_mesh,
        out_type=(jax.ShapeDtypeStruct((NPAD, HALF), jnp.float32),
                  jax.ShapeDtypeStruct((NPAD, HALF), jnp.float32)),
        scratch_types=[
            pltpu.VMEM((CHUNK,), jnp.int32),
            pltpu.VMEM((CHUNK,), jnp.int32),
            pltpu.VMEM((CHUNK,), jnp.int32),
            pltpu.VMEM((CHUNK,), jnp.int32),
            pltpu.VMEM((CHUNK, HALF), jnp.float32),
            pltpu.VMEM((CHUNK, HALF), jnp.float32),
            pltpu.VMEM((WCH,), jnp.int32),
            pltpu.VMEM((WCH, HALF), jnp.float32),
            pltpu.VMEM_SHARED((NPAD, HALF), jnp.float32),
            pltpu.SemaphoreType.DMA,
            pltpu.SemaphoreType.DMA,
            pltpu.SemaphoreType.DMA,
            pltpu.SemaphoreType.DMA,
            pltpu.SemaphoreType.DMA,
            pltpu.SemaphoreType.DMA,
        ],
    )
    def k(y0_h, y1_h, src_h, dst_h, ri_h, zr_h, out0_h, out1_h,
          isa, isb, ida, idb, rows_a, rows_b, ribuf, zob, acc_sh,
          s1, s2, s3, s4, sa, sb):
        c = lax.axis_index("c")
        s = lax.axis_index("s")
        ebase = s * EPS

        pltpu.sync_copy(zr_h, zob)
        for t in range(NWCH):
            pltpu.sync_copy(ri_h.at[pl.ds(s * ROWS_PER_SUB + t * WCH, WCH)],
                            ribuf)
            pltpu.sync_copy(zob, acc_sh.at[ribuf])
        plsc.subcore_barrier()

        def icp(buf, ref1d, j, sem):
            return pltpu.make_async_copy(
                ref1d.at[pl.ds(ebase + j * CHUNK, CHUNK)], buf, sem)

        def edge_loop(y_h):
            icp(isa, src_h, 0, s1).start()
            icp(ida, dst_h, 0, s2).start()

            @pl.loop(0, NWIN, step=2)
            def _(j):
                icp(isa, src_h, j, s1).wait()
                icp(ida, dst_h, j, s2).wait()
                ga = pltpu.async_copy(y_h.at[isa], rows_a, sa)
                icp(isb, src_h, j + 1, s3).start()
                icp(idb, dst_h, j + 1, s4).start()
                icp(isb, src_h, j + 1, s3).wait()
                icp(idb, dst_h, j + 1, s4).wait()
                gb = pltpu.async_copy(y_h.at[isb], rows_b, sb)
                ga.wait()
                pltpu.sync_copy(rows_a, acc_sh.at[ida], add=True)

                @pl.when(j + 2 < NWIN)
                def _():
                    icp(isa, src_h, j + 2, s1).start()
                    icp(ida, dst_h, j + 2, s2).start()

                gb.wait()
                pltpu.sync_copy(rows_b, acc_sh.at[idb], add=True)

        @pl.when(c == 0)
        def _():
            edge_loop(y0_h)

        @pl.when(c == 1)
        def _():
            edge_loop(y1_h)

        plsc.subcore_barrier()

        def readout(out_h):
            for t in range(NWCH):
                base = s * ROWS_PER_SUB + t * WCH
                pltpu.sync_copy(ri_h.at[pl.ds(base, WCH)], ribuf)
                pltpu.sync_copy(acc_sh.at[ribuf], zob)
                pltpu.sync_copy(zob, out_h.at[pl.ds(base, WCH)])

        @pl.when(c == 0)
        def _():
            readout(out0_h)

        @pl.when(c == 1)
        def _():
            readout(out1_h)

    return k(y0, y1, srcf, dstf, rif, zr)


@jax.jit
def _deghist(dstf, rif, zr, ones_rows):
    @functools.partial(
        pl.kernel,
        mesh=_mesh,
        out_type=(jax.ShapeDtypeStruct((NPAD, HALF), jnp.float32),
                  jax.ShapeDtypeStruct((NPAD, HALF), jnp.float32)),
        scratch_types=[
            pltpu.VMEM((DCH,), jnp.int32),
            pltpu.VMEM((DCH,), jnp.int32),
            pltpu.VMEM((DCH, HALF), jnp.float32),
            pltpu.VMEM((WCH,), jnp.int32),
            pltpu.VMEM((WCH, HALF), jnp.float32),
            pltpu.VMEM_SHARED((NPAD, HALF), jnp.float32),
            pltpu.SemaphoreType.DMA,
            pltpu.SemaphoreType.DMA,
            pltpu.SemaphoreType.DMA,
            pltpu.SemaphoreType.DMA,
        ],
    )
    def k(dst_h, ri_h, zr_h, or_h, out0_h, out1_h,
          ida, idb, orows, ribuf, zob, acc_sh, s1, s2, sa, sb):
        c = lax.axis_index("c")
        s = lax.axis_index("s")
        ebase = (s * NCORE + c) * DEPS

        pltpu.sync_copy(zr_h, zob)
        pltpu.sync_copy(or_h, orows)
        for t in range(NWCH):
            pltpu.sync_copy(ri_h.at[pl.ds(s * ROWS_PER_SUB + t * WCH, WCH)],
                            ribuf)
            pltpu.sync_copy(zob, acc_sh.at[ribuf])
        plsc.subcore_barrier()

        def icp(buf, j, sem):
            return pltpu.make_async_copy(
                dst_h.at[pl.ds(ebase + j * DCH, DCH)], buf, sem)

        icp(ida, 0, s1).start()
        icp(idb, 1, s2).start()

        @pl.loop(0, DNW, step=2)
        def _(j):
            icp(ida, j, s1).wait()
            ca = pltpu.async_copy(orows, acc_sh.at[ida], sa, add=True)
            icp(idb, j + 1, s2).wait()
            cb = pltpu.async_copy(orows, acc_sh.at[idb], sb, add=True)
            ca.wait()

            @pl.when(j + 2 < DNW)
            def _():
                icp(ida, j + 2, s1).start()

            cb.wait()

            @pl.when(j + 3 < DNW)
            def _():
                icp(idb, j + 3, s2).start()

        plsc.subcore_barrier()

        def readout(out_h):
            for t in range(NWCH):
                base = s * ROWS_PER_SUB + t * WCH
                pltpu.sync_copy(ri_h.at[pl.ds(base, WCH)], ribuf)
                pltpu.sync_copy(acc_sh.at[ribuf], zob)
                pltpu.sync_copy(zob, out_h.at[pl.ds(base, WCH)])

        @pl.when(c == 0)
        def _():
            readout(out0_h)

        @pl.when(c == 1)
        def _():
            readout(out1_h)

    return k(dstf, rif, zr, ones_rows)


def _stage_a_body(d0_ref, d1_ref, x_ref, w_ref, y0_ref, y1_ref, dinv_ref):
    deg = d0_ref[:, 0:1] + d1_ref[:, 0:1] + 1.0
    dinv = lax.rsqrt(deg)
    xw = jnp.dot(x_ref[...], w_ref[...], preferred_element_type=jnp.float32)
    yv = xw * dinv
    y0_ref[...] = yv[:, :HALF]
    y1_ref[...] = yv[:, HALF:]
    dinv_ref[...] = jnp.broadcast_to(dinv, (BLK, 16))


@jax.jit
def _stage_a(deg0, deg1, x, W1):
    return pl.pallas_call(
        _stage_a_body,
        grid=(NB,),
        in_specs=[
            pl.BlockSpec((BLK, HALF), lambda i: (i, 0)),
            pl.BlockSpec((BLK, HALF), lambda i: (i, 0)),
            pl.BlockSpec((BLK, IN_DIM), lambda i: (i, 0)),
            pl.BlockSpec((IN_DIM, HID_DIM), lambda i: (0, 0)),
        ],
        out_specs=[
            pl.BlockSpec((BLK, HALF), lambda i: (i, 0)),
            pl.BlockSpec((BLK, HALF), lambda i: (i, 0)),
            pl.BlockSpec((BLK, 16), lambda i: (i, 0)),
        ],
        out_shape=[
            jax.ShapeDtypeStruct((NPAD, HALF), jnp.float32),
            jax.ShapeDtypeStruct((NPAD, HALF), jnp.float32),
            jax.ShapeDtypeStruct((NPAD, 16), jnp.float32),
        ],
    )(deg0, deg1, x, W1)


def _stage_b_body(s0_ref, s1_ref, y0_ref, y1_ref, dinv_ref, b_ref, w_ref,
                  o0_ref, o1_ref):
    agg = jnp.concatenate([s0_ref[...], s1_ref[...]], axis=1)
    y = jnp.concatenate([y0_ref[...], y1_ref[...]], axis=1)
    dinv = dinv_ref[:, 0:1]
    h = jnp.maximum(dinv * (agg + y) + b_ref[...], 0.0)
    z = jnp.dot(h, w_ref[...], preferred_element_type=jnp.float32)
    y2 = z * dinv
    o0_ref[...] = y2[:, :HALF]
    o1_ref[...] = y2[:, HALF:]


@jax.jit
def _stage_b(S0, S1, y0, y1, dinv16, b1, W2):
    return pl.pallas_call(
        _stage_b_body,
        grid=(NB,),
        in_specs=[
            pl.BlockSpec((BLK, HALF), lambda i: (i, 0)),
            pl.BlockSpec((BLK, HALF), lambda i: (i, 0)),
            pl.BlockSpec((BLK, HALF), lambda i: (i, 0)),
            pl.BlockSpec((BLK, HALF), lambda i: (i, 0)),
            pl.BlockSpec((BLK, 16), lambda i: (i, 0)),
            pl.BlockSpec((1, HID_DIM), lambda i: (0, 0)),
            pl.BlockSpec((HID_DIM, HID_DIM), lambda i: (0, 0)),
        ],
        out_specs=[
            pl.BlockSpec((BLK, HALF), lambda i: (i, 0)),
            pl.BlockSpec((BLK, HALF), lambda i: (i, 0)),
        ],
        out_shape=[
            jax.ShapeDtypeStruct((NPAD, HALF), jnp.float32),
            jax.ShapeDtypeStruct((NPAD, HALF), jnp.float32),
        ],
    )(S0, S1, y0, y1, dinv16, b1, W2)


def _stage_c_body(s0_ref, s1_ref, y0_ref, y1_ref, dinv_ref, b_ref, batch_ref,
                  wp_ref, bp_ref, o_ref, sums, cnt):
    i = pl.program_id(0)

    @pl.when(i == 0)
    def _():
        sums[...] = jnp.zeros((N_GRAPHS, HID_DIM), jnp.float32)
        cnt[...] = jnp.zeros((N_GRAPHS, 128), jnp.float32)

    agg = jnp.concatenate([s0_ref[...], s1_ref[...]], axis=1)
    y = jnp.concatenate([y0_ref[...], y1_ref[...]], axis=1)
    dinv = dinv_ref[:, 0:1]
    h = jnp.maximum(dinv * (agg + y) + b_ref[...], 0.0)
    gids = lax.broadcasted_iota(jnp.int32, (N_GRAPHS, BLK), 0)
    mask = (batch_ref[0] == gids).astype(jnp.float32)
    sums[...] += jnp.dot(mask, h, preferred_element_type=jnp.float32)
    cnt[...] += jnp.broadcast_to(
        jnp.sum(mask, axis=1, keepdims=True), (N_GRAPHS, 128))

    @pl.when(i == NB - 1)
    def _():
        hg = sums[...] / jnp.maximum(cnt[:, 0:1], 1.0)
        o_ref[...] = (
            jnp.dot(hg, wp_ref[...], preferred_element_type=jnp.float32)
            + bp_ref[...])


@jax.jit
def _stage_c(S0, S1, y0, y1, dinv16, b2, batch3, Wp, bp):
    return pl.pallas_call(
        _stage_c_body,
        grid=(NB,),
        in_specs=[
            pl.BlockSpec((BLK, HALF), lambda i: (i, 0)),
            pl.BlockSpec((BLK, HALF), lambda i: (i, 0)),
            pl.BlockSpec((BLK, HALF), lambda i: (i, 0)),
            pl.BlockSpec((BLK, HALF), lambda i: (i, 0)),
            pl.BlockSpec((BLK, 16), lambda i: (i, 0)),
            pl.BlockSpec((1, HID_DIM), lambda i: (0, 0)),
            pl.BlockSpec((1, 1, BLK), lambda i: (i, 0, 0)),
            pl.BlockSpec((HID_DIM, OUT_DIM), lambda i: (0, 0)),
            pl.BlockSpec((1, OUT_DIM), lambda i: (0, 0)),
        ],
        out_specs=pl.BlockSpec((N_GRAPHS, OUT_DIM), lambda i: (0, 0)),
        out_shape=jax.ShapeDtypeStruct((N_GRAPHS, OUT_DIM), jnp.float32),
        scratch_shapes=[
            pltpu.VMEM((N_GRAPHS, HID_DIM), jnp.float32),
            pltpu.VMEM((N_GRAPHS, 128), jnp.float32),
        ],
    )(S0, S1, y0, y1, dinv16, b2, batch3, Wp, bp)


def kernel(x, edge_index, batch, W1, b1, W2, b2, Wp, bp):
    src = edge_index[0].astype(jnp.int32)
    dst = edge_index[1].astype(jnp.int32)
    xp = jnp.pad(x, ((0, NPAD - N_NODES), (0, 0)))
    batch3 = jnp.pad(batch.astype(jnp.int32), (0, NPAD - N_NODES),
                     constant_values=N_GRAPHS).reshape(NB, 1, BLK)

    rif = jnp.arange(NPAD, dtype=jnp.int32)
    zr = jnp.zeros((WCH, HALF), jnp.float32)
    ones_rows = jnp.ones((DCH, HALF), jnp.float32)

    deg0, deg1 = _deghist(dst, rif, zr, ones_rows)
    y0, y1, dinv16 = _stage_a(deg0, deg1, xp, W1)
    S0, S1 = _segsum(y0, y1, src, dst, rif, zr)
    z0, z1 = _stage_b(S0, S1, y0, y1, dinv16, b1.reshape(1, HID_DIM), W2)
    T0, T1 = _segsum(z0, z1, src, dst, rif, zr)
    return _stage_c(T0, T1, z0, z1, dinv16, b2.reshape(1, HID_DIM), batch3,
                    Wp, bp.reshape(1, OUT_DIM))
